# baseline (speedup 1.0000x reference)
import functools

import jax
import jax.numpy as jnp
from jax import lax
from jax.experimental import pallas as pl
from jax.experimental.pallas import tpu as pltpu
from jax.experimental.pallas import tpu_sc as plsc

_N_ROWS = 4096 * 200
_D = 128
_NC, _NS = 2, 16
_NW = _NC * _NS
_ROWS_PER_W = _N_ROWS // _NW
_CHUNK = 256
_NIT = _ROWS_PER_W // _CHUNK
_GPC = _CHUNK // 128


_mesh = plsc.VectorSubcoreMesh(core_axis_name="c", subcore_axis_name="s")


@functools.partial(
    pl.kernel,
    mesh=_mesh,
    out_type=jax.ShapeDtypeStruct((_N_ROWS, _D), jnp.float32),
    scratch_types=[
        pltpu.VMEM((_ROWS_PER_W,), jnp.int32),
        pltpu.VMEM((2, _CHUNK, _D), jnp.float32),
        pltpu.SemaphoreType.DMA,
        pltpu.SemaphoreType.DMA,
        pltpu.SemaphoreType.DMA,
        pltpu.SemaphoreType.DMA,
    ],
)
def _gather_rows(idx_hbm, table_hbm, out_hbm, idx_v, rows_v, gs0, gs1, ss0, ss1):
    wid = lax.axis_index("s") * _NC + lax.axis_index("c")
    base = wid * _ROWS_PER_W
    pltpu.sync_copy(idx_hbm.at[pl.ds(base, _ROWS_PER_W)], idx_v)

    gsem = (gs0, gs1)
    ssem = (ss0, ss1)

    def fire_gather(it, b):
        for j in range(_GPC):
            pltpu.async_copy(
                table_hbm.at[idx_v.at[pl.ds(it * _CHUNK + j * 128, 128)]],
                rows_v.at[b, pl.ds(j * 128, 128)],
                gsem[b],
            )

    def wait_gather(b):
        pltpu.make_async_copy(
            out_hbm.at[pl.ds(0, _CHUNK)], rows_v.at[b], gsem[b]
        ).wait()

    def fire_store(it, b):
        pltpu.async_copy(
            rows_v.at[b], out_hbm.at[pl.ds(base + it * _CHUNK, _CHUNK)], ssem[b]
        )

    def wait_store(b):
        pltpu.make_async_copy(
            rows_v.at[b], out_hbm.at[pl.ds(0, _CHUNK)], ssem[b]
        ).wait()

    fire_gather(0, 0)
    wait_gather(0)
    fire_store(0, 0)
    fire_gather(1, 1)
    wait_gather(1)
    fire_store(1, 1)

    def pair(p, _):
        it0 = 2 * p
        wait_store(0)
        fire_gather(it0, 0)
        wait_gather(0)
        fire_store(it0, 0)
        wait_store(1)
        fire_gather(it0 + 1, 1)
        wait_gather(1)
        fire_store(it0 + 1, 1)
        return ()

    lax.fori_loop(1, _NIT // 2, pair, ())
    wait_store(0)
    wait_store(1)


def kernel(x, table):
    idx = x.reshape(_N_ROWS).astype(jnp.int32)
    out = _gather_rows(idx, table)
    return out.reshape(x.shape[0], x.shape[1], _D)


# speedup vs baseline: 13.8162x; 13.8162x over previous
import functools

import jax
import jax.numpy as jnp
from jax import lax
from jax.experimental import pallas as pl
from jax.experimental.pallas import tpu as pltpu
from jax.experimental.pallas import tpu_sc as plsc

_N_ROWS = 4096 * 200
_D = 128
_NC, _NS = 2, 16
_NW = _NC * _NS
_ROWS_PER_W = _N_ROWS // _NW
_CHUNK = 256
_NIT = _ROWS_PER_W // _CHUNK
_GPC = _CHUNK // 128
_K = 128
_NSL = _ROWS_PER_W // 16


_mesh = plsc.VectorSubcoreMesh(core_axis_name="c", subcore_axis_name="s")


@functools.partial(
    pl.kernel,
    mesh=_mesh,
    out_type=jax.ShapeDtypeStruct((_N_ROWS, _D), jnp.float32),
    scratch_types=[
        pltpu.VMEM((_ROWS_PER_W,), jnp.int32),
        pltpu.VMEM((2, _CHUNK, _D), jnp.float32),
        pltpu.SemaphoreType.DMA,
        pltpu.SemaphoreType.DMA,
        pltpu.SemaphoreType.DMA,
        pltpu.SemaphoreType.DMA,
    ],
)
def _gather_rows(idx_hbm, table_hbm, out_hbm, idx_v, rows_v, gs0, gs1, ss0, ss1):
    wid = lax.axis_index("s") * _NC + lax.axis_index("c")
    base = wid * _ROWS_PER_W
    pltpu.sync_copy(idx_hbm.at[pl.ds(base, _ROWS_PER_W)], idx_v)

    lane_off = 3 * lax.iota(jnp.int32, 16)

    def spread(s, _):
        rep = jnp.full((16,), 3 * 16 * lax.rem(s, 8), dtype=jnp.int32)
        sl = pl.ds(s * 16, 16)
        idx_v[sl] = idx_v[sl] + lane_off + rep
        return ()

    lax.fori_loop(0, _NSL, spread, ())

    gsem = (gs0, gs1)
    ssem = (ss0, ss1)

    def fire_gather(it, b):
        for j in range(_GPC):
            pltpu.async_copy(
                table_hbm.at[idx_v.at[pl.ds(it * _CHUNK + j * 128, 128)]],
                rows_v.at[b, pl.ds(j * 128, 128)],
                gsem[b],
            )

    def wait_gather(b):
        pltpu.make_async_copy(
            out_hbm.at[pl.ds(0, _CHUNK)], rows_v.at[b], gsem[b]
        ).wait()

    def fire_store(it, b):
        pltpu.async_copy(
            rows_v.at[b], out_hbm.at[pl.ds(base + it * _CHUNK, _CHUNK)], ssem[b]
        )

    def wait_store(b):
        pltpu.make_async_copy(
            rows_v.at[b], out_hbm.at[pl.ds(0, _CHUNK)], ssem[b]
        ).wait()

    fire_gather(0, 0)
    wait_gather(0)
    fire_store(0, 0)
    fire_gather(1, 1)
    wait_gather(1)
    fire_store(1, 1)

    def pair(p, _):
        it0 = 2 * p
        wait_store(0)
        fire_gather(it0, 0)
        wait_gather(0)
        fire_store(it0, 0)
        wait_store(1)
        fire_gather(it0 + 1, 1)
        wait_gather(1)
        fire_store(it0 + 1, 1)
        return ()

    lax.fori_loop(1, _NIT // 2, pair, ())
    wait_store(0)
    wait_store(1)


def kernel(x, table):
    idx = x.reshape(_N_ROWS).astype(jnp.int32)
    rep_table = jnp.tile(table, (_K, 1))
    out = _gather_rows(idx, rep_table)
    return out.reshape(x.shape[0], x.shape[1], _D)


# speedup vs baseline: 29.2669x; 2.1183x over previous
import functools

import jax
import jax.numpy as jnp
from jax import lax
from jax.experimental import pallas as pl
from jax.experimental.pallas import tpu as pltpu
from jax.experimental.pallas import tpu_sc as plsc

_N_ROWS = 4096 * 200
_D = 128
_NC, _NS = 2, 16
_NW = _NC * _NS
_ROWS_PER_W = _N_ROWS // _NW
_CHUNK = 256
_NIT = _ROWS_PER_W // _CHUNK
_GPC = _CHUNK // 128
_K = 1024
_NSL = _ROWS_PER_W // 16


_mesh = plsc.VectorSubcoreMesh(core_axis_name="c", subcore_axis_name="s")


@functools.partial(
    pl.kernel,
    mesh=_mesh,
    out_type=jax.ShapeDtypeStruct((_N_ROWS, _D), jnp.float32),
    scratch_types=[
        pltpu.VMEM((_ROWS_PER_W,), jnp.int32),
        pltpu.VMEM((2, _CHUNK, _D), jnp.float32),
        pltpu.SemaphoreType.DMA,
        pltpu.SemaphoreType.DMA,
        pltpu.SemaphoreType.DMA,
        pltpu.SemaphoreType.DMA,
    ],
)
def _gather_rows(idx_hbm, table_hbm, out_hbm, idx_v, rows_v, gs0, gs1, ss0, ss1):
    wid = lax.axis_index("s") * _NC + lax.axis_index("c")
    base = wid * _ROWS_PER_W
    pltpu.sync_copy(idx_hbm.at[pl.ds(base, _ROWS_PER_W)], idx_v)

    lane_off = 3 * lax.iota(jnp.int32, 16)

    def spread(s, _):
        rep = jnp.full((16,), 3 * 16 * lax.rem(s, _K // 16), dtype=jnp.int32)
        sl = pl.ds(s * 16, 16)
        idx_v[sl] = idx_v[sl] + lane_off + rep
        return ()

    lax.fori_loop(0, _NSL, spread, ())

    gsem = (gs0, gs1)
    ssem = (ss0, ss1)

    def fire_gather(it, b):
        for j in range(_GPC):
            pltpu.async_copy(
                table_hbm.at[idx_v.at[pl.ds(it * _CHUNK + j * 128, 128)]],
                rows_v.at[b, pl.ds(j * 128, 128)],
                gsem[b],
            )

    def wait_gather(b):
        pltpu.make_async_copy(
            out_hbm.at[pl.ds(0, _CHUNK)], rows_v.at[b], gsem[b]
        ).wait()

    def fire_store(it, b):
        pltpu.async_copy(
            rows_v.at[b], out_hbm.at[pl.ds(base + it * _CHUNK, _CHUNK)], ssem[b]
        )

    def wait_store(b):
        pltpu.make_async_copy(
            rows_v.at[b], out_hbm.at[pl.ds(0, _CHUNK)], ssem[b]
        ).wait()

    fire_gather(0, 0)
    wait_gather(0)
    fire_store(0, 0)
    fire_gather(1, 1)
    wait_gather(1)
    fire_store(1, 1)

    def pair(p, _):
        it0 = 2 * p
        wait_store(0)
        fire_gather(it0, 0)
        wait_gather(0)
        fire_store(it0, 0)
        wait_store(1)
        fire_gather(it0 + 1, 1)
        wait_gather(1)
        fire_store(it0 + 1, 1)
        return ()

    lax.fori_loop(1, _NIT // 2, pair, ())
    wait_store(0)
    wait_store(1)


def kernel(x, table):
    idx = x.reshape(_N_ROWS).astype(jnp.int32)
    rep_table = jnp.tile(table, (_K, 1))
    out = _gather_rows(idx, rep_table)
    return out.reshape(x.shape[0], x.shape[1], _D)


# speedup vs baseline: 31.2763x; 1.0687x over previous
import functools

import jax
import jax.numpy as jnp
from jax import lax
from jax.experimental import pallas as pl
from jax.experimental.pallas import tpu as pltpu
from jax.experimental.pallas import tpu_sc as plsc

_N_ROWS = 4096 * 200
_D = 128
_NC, _NS = 2, 16
_NW = _NC * _NS
_ROWS_PER_W = _N_ROWS // _NW
_CHUNK = 256
_NIT = _ROWS_PER_W // _CHUNK
_GPC = _CHUNK // 128
_K = 4096
_NSL = _ROWS_PER_W // 16


_mesh = plsc.VectorSubcoreMesh(core_axis_name="c", subcore_axis_name="s")


@functools.partial(
    pl.kernel,
    mesh=_mesh,
    out_type=jax.ShapeDtypeStruct((_N_ROWS, _D), jnp.float32),
    scratch_types=[
        pltpu.VMEM((_ROWS_PER_W,), jnp.int32),
        pltpu.VMEM((2, _CHUNK, _D), jnp.float32),
        pltpu.SemaphoreType.DMA,
        pltpu.SemaphoreType.DMA,
        pltpu.SemaphoreType.DMA,
        pltpu.SemaphoreType.DMA,
    ],
)
def _gather_rows(idx_hbm, table_hbm, out_hbm, idx_v, rows_v, gs0, gs1, ss0, ss1):
    wid = lax.axis_index("s") * _NC + lax.axis_index("c")
    base = wid * _ROWS_PER_W
    pltpu.sync_copy(idx_hbm.at[pl.ds(base, _ROWS_PER_W)], idx_v)

    lane_off = 3 * lax.iota(jnp.int32, 16)

    phase = wid * (_K // 16 // _NW)

    def spread(s, _):
        rep = jnp.full(
            (16,), 3 * 16 * lax.rem(s + phase, _K // 16), dtype=jnp.int32
        )
        sl = pl.ds(s * 16, 16)
        idx_v[sl] = idx_v[sl] + lane_off + rep
        return ()

    lax.fori_loop(0, _NSL, spread, ())

    gsem = (gs0, gs1)
    ssem = (ss0, ss1)

    def fire_gather(it, b):
        for j in range(_GPC):
            pltpu.async_copy(
                table_hbm.at[idx_v.at[pl.ds(it * _CHUNK + j * 128, 128)]],
                rows_v.at[b, pl.ds(j * 128, 128)],
                gsem[b],
            )

    def wait_gather(b):
        pltpu.make_async_copy(
            out_hbm.at[pl.ds(0, _CHUNK)], rows_v.at[b], gsem[b]
        ).wait()

    def fire_store(it, b):
        pltpu.async_copy(
            rows_v.at[b], out_hbm.at[pl.ds(base + it * _CHUNK, _CHUNK)], ssem[b]
        )

    def wait_store(b):
        pltpu.make_async_copy(
            rows_v.at[b], out_hbm.at[pl.ds(0, _CHUNK)], ssem[b]
        ).wait()

    fire_gather(0, 0)
    wait_gather(0)
    fire_store(0, 0)
    fire_gather(1, 1)
    wait_gather(1)
    fire_store(1, 1)

    def pair(p, _):
        it0 = 2 * p
        wait_store(0)
        fire_gather(it0, 0)
        wait_gather(0)
        fire_store(it0, 0)
        wait_store(1)
        fire_gather(it0 + 1, 1)
        wait_gather(1)
        fire_store(it0 + 1, 1)
        return ()

    lax.fori_loop(1, _NIT // 2, pair, ())
    wait_store(0)
    wait_store(1)


def kernel(x, table):
    idx = x.reshape(_N_ROWS).astype(jnp.int32)
    rep_table = jnp.tile(table, (_K, 1))
    out = _gather_rows(idx, rep_table)
    return out.reshape(x.shape[0], x.shape[1], _D)
